# baseline (speedup 1.0000x reference)
import functools

import jax
import jax.numpy as jnp
from jax import lax
from jax.experimental import pallas as pl
from jax.experimental.pallas import tpu as pltpu
from jax.experimental.pallas import tpu_sc as plsc

NUM_QUANTS = 512
DIM_VEC = 256
PACKED = DIM_VEC // 2
LANES = 16
CHUNK = 128
NWORKERS = 32


def kernel(y, emb_table):
    n_rows, n_cols = y.shape
    batch = n_rows * n_cols
    per_w = batch // NWORKERS
    nsteps = per_w // CHUNK
    y_flat = y.reshape(batch)
    bits = lax.bitcast_convert_type(emb_table.astype(jnp.bfloat16), jnp.uint16)
    a = bits.reshape(NUM_QUANTS, DIM_VEC // 32, 2, LANES).astype(jnp.uint32)
    tab_packed = lax.bitcast_convert_type(
        a[:, :, 0, :] | (a[:, :, 1, :] << 16), jnp.int32
    ).reshape(NUM_QUANTS * PACKED)

    mesh = plsc.VectorSubcoreMesh(core_axis_name="c", subcore_axis_name="s")

    @functools.partial(
        pl.kernel,
        mesh=mesh,
        out_type=jax.ShapeDtypeStruct((batch, PACKED), jnp.int32),
        scratch_types=[
            pltpu.VMEM((per_w,), jnp.float32),
            pltpu.VMEM((per_w,), jnp.int32),
            pltpu.VMEM((NUM_QUANTS * PACKED,), jnp.int32),
            pltpu.VMEM((2, CHUNK, PACKED), jnp.int32),
            pltpu.SemaphoreType.DMA,
            pltpu.SemaphoreType.DMA((2,)),
        ],
    )
    def sc_lookup(y_hbm, tabp_hbm, out_hbm, y_v, idx_v, tab_v, stage_v,
                  tsem, ssem):
        wid = lax.axis_index("s") * 2 + lax.axis_index("c")
        base = wid * per_w

        tab_copy = pltpu.make_async_copy(tabp_hbm, tab_v, tsem)
        tab_copy.start()

        pltpu.sync_copy(y_hbm.at[pl.ds(base, per_w)], y_v)

        @pl.loop(0, per_w, step=LANES)
        def _(j):
            sl = pl.ds(j, LANES)
            yc = jnp.minimum(jnp.maximum(y_v[sl], -1.0), 1.0)
            t = (yc + 1.0) * 0.5 * float(NUM_QUANTS - 1)
            idx_v[sl] = t.astype(jnp.int32) * PACKED

        tab_copy.wait()

        def build_chunk(s, b):
            @pl.loop(0, CHUNK, step=LANES)
            def _(g):
                vrow = idx_v[pl.ds(s * CHUNK + g, LANES)]
                for l in range(LANES):
                    row = vrow[l]
                    for u in range(PACKED // LANES):
                        stage_v[b, g + l, pl.ds(u * LANES, LANES)] = (
                            tab_v[pl.ds(row + u * LANES, LANES)])

        def start_scatter(s, b):
            pltpu.async_copy(
                stage_v.at[b],
                out_hbm.at[pl.ds(base + s * CHUNK, CHUNK)],
                ssem.at[b],
            )

        def wait_scatter(b):
            pltpu.make_async_copy(
                stage_v.at[b],
                out_hbm.at[pl.ds(base, CHUNK)],
                ssem.at[b],
            ).wait()

        @pl.loop(0, nsteps, step=2)
        def _(i):
            for b in (0, 1):
                s = i + b
                @pl.when(s >= 2)
                def _():
                    wait_scatter(b)
                build_chunk(s, b)
                start_scatter(s, b)

        wait_scatter(0)
        wait_scatter(1)

    packed_rows = sc_lookup(y_flat, tab_packed)
    lo = lax.bitcast_convert_type(
        lax.shift_left(packed_rows, jnp.int32(16)), jnp.float32)
    hi = lax.bitcast_convert_type(
        lax.bitwise_and(packed_rows, jnp.int32(-65536)), jnp.float32)
    out = jnp.stack(
        [lo.reshape(batch, DIM_VEC // 32, LANES),
         hi.reshape(batch, DIM_VEC // 32, LANES)],
        axis=2,
    ).reshape(n_rows, n_cols, DIM_VEC)
    return out


# speedup vs baseline: 4.5125x; 4.5125x over previous
import functools

import jax
import jax.numpy as jnp
from jax import lax
from jax.experimental import pallas as pl
from jax.experimental.pallas import tpu as pltpu
from jax.experimental.pallas import tpu_sc as plsc

NUM_QUANTS = 512
DIM_VEC = 256
PACKED = DIM_VEC // 2
LANES = 16
CHUNK = 64
NWORKERS = 32


def _pack_table(emb_table):
    bits = lax.bitcast_convert_type(emb_table.astype(jnp.bfloat16), jnp.uint16)
    a = bits.reshape(NUM_QUANTS, DIM_VEC // 32, 2, LANES).astype(jnp.uint32)
    packed = a[:, :, 0, :] | (a[:, :, 1, :] << 16)
    return lax.bitcast_convert_type(packed, jnp.int32).reshape(
        NUM_QUANTS * PACKED)


def kernel(y, emb_table):
    n_rows, n_cols = y.shape
    batch = n_rows * n_cols
    per_w = batch // NWORKERS
    nsteps = per_w // CHUNK
    y_flat = y.reshape(batch)
    tab_packed = _pack_table(emb_table)

    mesh = plsc.VectorSubcoreMesh(core_axis_name="c", subcore_axis_name="s")

    @functools.partial(
        pl.kernel,
        mesh=mesh,
        compiler_params=pltpu.CompilerParams(use_tc_tiling_on_sc=True),
        out_type=jax.ShapeDtypeStruct((batch, DIM_VEC), jnp.float32),
        scratch_types=[
            pltpu.VMEM((per_w,), jnp.float32),
            pltpu.VMEM((per_w,), jnp.int32),
            pltpu.VMEM((NUM_QUANTS * PACKED,), jnp.int32),
            pltpu.VMEM((2, CHUNK, DIM_VEC), jnp.float32),
            pltpu.SemaphoreType.DMA,
            pltpu.SemaphoreType.DMA((2,)),
        ],
    )
    def sc_lookup(y_hbm, tabp_hbm, out_hbm, y_v, idx_v, tab_v, stage_v,
                  tsem, ssem):
        wid = lax.axis_index("s") * 2 + lax.axis_index("c")
        base = wid * per_w

        tab_copy = pltpu.make_async_copy(tabp_hbm, tab_v, tsem)
        tab_copy.start()

        pltpu.sync_copy(y_hbm.at[pl.ds(base, per_w)], y_v)

        @pl.loop(0, per_w, step=LANES)
        def _(j):
            sl = pl.ds(j, LANES)
            yc = jnp.minimum(jnp.maximum(y_v[sl], -1.0), 1.0)
            t = (yc + 1.0) * 0.5 * float(NUM_QUANTS - 1)
            idx_v[sl] = t.astype(jnp.int32) * PACKED

        tab_copy.wait()

        hi_mask = jnp.int32(-65536)

        def build_chunk(s, b):
            @pl.loop(0, CHUNK, step=LANES)
            def _(g):
                vrow = idx_v[pl.ds(s * CHUNK + g, LANES)]
                for l in range(LANES):
                    row = vrow[l]
                    for u in range(DIM_VEC // 32):
                        v = tab_v[pl.ds(row + u * LANES, LANES)]
                        lo = lax.bitcast_convert_type(
                            lax.shift_left(v, jnp.int32(16)), jnp.float32)
                        hi = lax.bitcast_convert_type(
                            lax.bitwise_and(v, hi_mask), jnp.float32)
                        stage_v[b, g + l, pl.ds(u * 32, LANES)] = lo
                        stage_v[b, g + l, pl.ds(u * 32 + LANES, LANES)] = hi

        def start_scatter(s, b):
            pltpu.async_copy(
                stage_v.at[b],
                out_hbm.at[pl.ds(base + s * CHUNK, CHUNK)],
                ssem.at[b],
            )

        def wait_scatter(b):
            pltpu.make_async_copy(
                stage_v.at[b],
                out_hbm.at[pl.ds(base, CHUNK)],
                ssem.at[b],
            ).wait()

        @pl.loop(0, nsteps, step=2)
        def _(i):
            for b in (0, 1):
                s = i + b
                @pl.when(s >= 2)
                def _():
                    wait_scatter(b)
                build_chunk(s, b)
                start_scatter(s, b)

        wait_scatter(0)
        wait_scatter(1)

    out = sc_lookup(y_flat, tab_packed)
    return out.reshape(n_rows, n_cols, DIM_VEC)


# speedup vs baseline: 5.0800x; 1.1258x over previous
import functools

import jax
import jax.numpy as jnp
from jax import lax
from jax.experimental import pallas as pl
from jax.experimental.pallas import tpu as pltpu
from jax.experimental.pallas import tpu_sc as plsc

NUM_QUANTS = 512
DIM_VEC = 256
PACKED = DIM_VEC // 2
LANES = 16
CHUNK = 64
NWORKERS = 32


def _pack_table(emb_table):
    bits = lax.bitcast_convert_type(emb_table.astype(jnp.bfloat16), jnp.uint16)
    a = bits.reshape(NUM_QUANTS, DIM_VEC // 32, 2, LANES).astype(jnp.uint32)
    packed = a[:, :, 0, :] | (a[:, :, 1, :] << 16)
    return lax.bitcast_convert_type(packed, jnp.int32).reshape(
        NUM_QUANTS * PACKED)


def kernel(y, emb_table):
    n_rows, n_cols = y.shape
    batch = n_rows * n_cols
    per_w = batch // NWORKERS
    nsteps = per_w // CHUNK
    y_flat = y.reshape(batch)
    tab_packed = _pack_table(emb_table)

    mesh = plsc.VectorSubcoreMesh(core_axis_name="c", subcore_axis_name="s")

    samp_per_w = n_rows // NWORKERS
    SAMP = 2
    nsteps_s = samp_per_w // SAMP
    rows_per_chunk = SAMP * n_cols

    @functools.partial(
        pl.kernel,
        mesh=mesh,
        compiler_params=pltpu.CompilerParams(use_tc_tiling_on_sc=True),
        out_type=jax.ShapeDtypeStruct((n_rows, n_cols, DIM_VEC), jnp.float32),
        scratch_types=[
            pltpu.VMEM((per_w + LANES,), jnp.float32),
            pltpu.VMEM((per_w + LANES,), jnp.int32),
            pltpu.VMEM((NUM_QUANTS * PACKED,), jnp.int32),
            pltpu.VMEM((2, SAMP, n_cols, DIM_VEC), jnp.float32),
            pltpu.SemaphoreType.DMA,
            pltpu.SemaphoreType.DMA((2,)),
        ],
    )
    def sc_lookup(y_hbm, tabp_hbm, out_hbm, y_v, idx_v, tab_v, stage_v,
                  tsem, ssem):
        wid = lax.axis_index("s") * 2 + lax.axis_index("c")
        base = wid * per_w
        samp_base = wid * samp_per_w

        tab_copy = pltpu.make_async_copy(tabp_hbm, tab_v, tsem)
        tab_copy.start()

        pltpu.sync_copy(y_hbm.at[pl.ds(base, per_w)], y_v.at[pl.ds(0, per_w)])

        @pl.loop(0, per_w, step=LANES)
        def _(j):
            sl = pl.ds(j, LANES)
            yc = jnp.minimum(jnp.maximum(y_v[sl], -1.0), 1.0)
            t = (yc + 1.0) * 0.5 * float(NUM_QUANTS - 1)
            idx_v[sl] = t.astype(jnp.int32) * PACKED

        tab_copy.wait()

        hi_mask = jnp.int32(-65536)

        def build_chunk(s, b):
            for gbase in range(0, rows_per_chunk, LANES):
                vrow = idx_v[pl.ds(s * rows_per_chunk + gbase, LANES)]
                for l in range(min(LANES, rows_per_chunk - gbase)):
                    row = vrow[l]
                    r = gbase + l
                    for u in range(DIM_VEC // 32):
                        v = tab_v[pl.ds(row + u * LANES, LANES)]
                        lo = lax.bitcast_convert_type(
                            lax.shift_left(v, jnp.int32(16)), jnp.float32)
                        hi = lax.bitcast_convert_type(
                            lax.bitwise_and(v, hi_mask), jnp.float32)
                        stage_v[b, r // n_cols, r % n_cols,
                                pl.ds(u * 32, LANES)] = lo
                        stage_v[b, r // n_cols, r % n_cols,
                                pl.ds(u * 32 + LANES, LANES)] = hi

        def start_scatter(s, b):
            pltpu.async_copy(
                stage_v.at[b],
                out_hbm.at[pl.ds(samp_base + s * SAMP, SAMP)],
                ssem.at[b],
            )

        def wait_scatter(b):
            pltpu.make_async_copy(
                stage_v.at[b],
                out_hbm.at[pl.ds(samp_base, SAMP)],
                ssem.at[b],
            ).wait()

        @pl.loop(0, nsteps_s, step=2)
        def _(i):
            for b in (0, 1):
                s = i + b
                @pl.when(s >= 2)
                def _():
                    wait_scatter(b)
                build_chunk(s, b)
                start_scatter(s, b)

        wait_scatter(0)
        wait_scatter(1)

    return sc_lookup(y_flat, tab_packed)
